# Initial kernel scaffold; baseline (speedup 1.0000x reference)
import jax
import jax.numpy as jnp
from jax.experimental import pallas as pl


def kernel(x, projector, codebook):
    raise NotImplementedError("write your pallas kernel here")


# speedup vs baseline: 1.9068x; 1.9068x over previous
import numpy as np
import jax
import jax.numpy as jnp
from jax.experimental import pallas as pl
from jax.experimental.pallas import tpu as pltpu

_N = 256
_K = 33
_V = 64
_CB = 1024
_R = 1024
_ROWS = 4 * 4 * 1024

_BR_C = np.fft.rfft(np.eye(_N), norm='ortho')[:, :_K]
_BRR = jnp.asarray(_BR_C.real, dtype=jnp.float32)
_BRI = jnp.asarray(_BR_C.imag, dtype=jnp.float32)
_BRRT = jnp.asarray(_BR_C.real.T.copy(), dtype=jnp.float32)
_BRIT = jnp.asarray(_BR_C.imag.T.copy(), dtype=jnp.float32)
_CR = jnp.asarray(np.fft.irfft(np.eye(_K), n=_V, norm='ortho'),
                  dtype=jnp.float32)
_CI = jnp.asarray(np.fft.irfft(1j * np.eye(_K), n=_V, norm='ortho'),
                  dtype=jnp.float32)

_HI = jax.lax.Precision.HIGHEST


def _vq_kernel(x_ref, p_ref, cb_ref, brr_ref, bri_ref, brrt_ref, brit_ref,
               cr_ref, ci_ref, out_ref):
    p = p_ref[...]
    pr = jnp.dot(brrt_ref[...], p, precision=_HI)
    pi = jnp.dot(brit_ref[...], p, precision=_HI)
    cr = cr_ref[...]
    ci = ci_ref[...]
    d1 = pr * cr + pi * ci
    d2 = pr * ci - pi * cr
    m = (jnp.dot(brr_ref[...], d1, precision=_HI)
         + jnp.dot(bri_ref[...], d2, precision=_HI))

    cb = cb_ref[...]
    cbn = cb / jnp.sqrt(jnp.sum(cb * cb, axis=1, keepdims=True))

    f = jnp.dot(x_ref[...], m, precision=_HI)
    f = f / jnp.sqrt(jnp.sum(f * f, axis=1, keepdims=True))
    sim = jax.lax.dot_general(f, cbn, (((1,), (1,)), ((), ())))
    idx = jnp.argmax(sim, axis=1).astype(jnp.int32)
    out_ref[...] = idx.reshape(1, 1, _R)


def kernel(x, projector, codebook):
    x2 = x.reshape(_ROWS, _N)
    p2 = projector.reshape(_N, 1)
    nblk = _ROWS // _R
    full = lambda shape: pl.BlockSpec(shape, lambda i: (0,) * len(shape))
    out = pl.pallas_call(
        _vq_kernel,
        grid=(nblk,),
        in_specs=[
            pl.BlockSpec((_R, _N), lambda i: (i, 0)),
            full((_N, 1)),
            full((_CB, _V)),
            full((_N, _K)),
            full((_N, _K)),
            full((_K, _N)),
            full((_K, _N)),
            full((_K, _V)),
            full((_K, _V)),
        ],
        out_specs=pl.BlockSpec((1, 1, _R), lambda i: (i, 0, 0)),
        out_shape=jax.ShapeDtypeStruct((nblk, 1, _R), jnp.int32),
        compiler_params=pltpu.CompilerParams(
            dimension_semantics=("arbitrary",)),
    )(x2, p2, codebook, _BRR, _BRI, _BRRT, _BRIT, _CR, _CI)
    return out.reshape(x.shape[:-1])


# speedup vs baseline: 2.4394x; 1.2793x over previous
import numpy as np
import jax
import jax.numpy as jnp
from jax.experimental import pallas as pl
from jax.experimental.pallas import tpu as pltpu

_N = 256
_K = 33
_V = 64
_CB = 1024
_R = 1024
_ROWS = 4 * 4 * 1024

_BR_C = np.fft.rfft(np.eye(_N), norm='ortho')[:, :_K]
_BRR = jnp.asarray(_BR_C.real, dtype=jnp.float32)
_BRI = jnp.asarray(_BR_C.imag, dtype=jnp.float32)
_BRRT = jnp.asarray(_BR_C.real.T.copy(), dtype=jnp.float32)
_BRIT = jnp.asarray(_BR_C.imag.T.copy(), dtype=jnp.float32)
_CR = jnp.asarray(np.fft.irfft(np.eye(_K), n=_V, norm='ortho'),
                  dtype=jnp.float32)
_CI = jnp.asarray(np.fft.irfft(1j * np.eye(_K), n=_V, norm='ortho'),
                  dtype=jnp.float32)

_HI = jax.lax.Precision.HIGHEST


def _vq_kernel(x_ref, p_ref, cb_ref, brr_ref, bri_ref, brrt_ref, brit_ref,
               cr_ref, ci_ref, out_ref, m_ref, cbn_ref):
    @pl.when(pl.program_id(0) == 0)
    def _init():
        p = p_ref[...]
        pr = jnp.dot(brrt_ref[...], p, precision=_HI)
        pi = jnp.dot(brit_ref[...], p, precision=_HI)
        cr = cr_ref[...]
        ci = ci_ref[...]
        d1 = pr * cr + pi * ci
        d2 = pr * ci - pi * cr
        m_ref[...] = (jnp.dot(brr_ref[...], d1, precision=_HI)
                      + jnp.dot(bri_ref[...], d2, precision=_HI))
        cb = cb_ref[...]
        cbn_ref[...] = cb / jnp.sqrt(jnp.sum(cb * cb, axis=1, keepdims=True))

    f = jnp.dot(x_ref[...], m_ref[...], precision=_HI)
    f = f / jnp.sqrt(jnp.sum(f * f, axis=1, keepdims=True))
    sim = jax.lax.dot_general(f, cbn_ref[...], (((1,), (1,)), ((), ())))
    idx = jnp.argmax(sim, axis=1).astype(jnp.int32)
    out_ref[...] = idx.reshape(1, 1, _R)


def kernel(x, projector, codebook):
    x2 = x.reshape(_ROWS, _N)
    p2 = projector.reshape(_N, 1)
    nblk = _ROWS // _R
    full = lambda shape: pl.BlockSpec(shape, lambda i: (0,) * len(shape))
    out = pl.pallas_call(
        _vq_kernel,
        grid=(nblk,),
        in_specs=[
            pl.BlockSpec((_R, _N), lambda i: (i, 0)),
            full((_N, 1)),
            full((_CB, _V)),
            full((_N, _K)),
            full((_N, _K)),
            full((_K, _N)),
            full((_K, _N)),
            full((_K, _V)),
            full((_K, _V)),
        ],
        out_specs=pl.BlockSpec((1, 1, _R), lambda i: (i, 0, 0)),
        out_shape=jax.ShapeDtypeStruct((nblk, 1, _R), jnp.int32),
        scratch_shapes=[
            pltpu.VMEM((_N, _V), jnp.float32),
            pltpu.VMEM((_CB, _V), jnp.float32),
        ],
        compiler_params=pltpu.CompilerParams(
            dimension_semantics=("arbitrary",)),
    )(x2, p2, codebook, _BRR, _BRI, _BRRT, _BRIT, _CR, _CI)
    return out.reshape(x.shape[:-1])
